# Initial kernel scaffold; baseline (speedup 1.0000x reference)
import jax
import jax.numpy as jnp
from jax.experimental import pallas as pl


def kernel(logits, targets):
    raise NotImplementedError("write your pallas kernel here")


# speedup vs baseline: 2.7669x; 2.7669x over previous
import functools
import math

import jax
import jax.numpy as jnp
from jax import lax
from jax.experimental import pallas as pl
from jax.experimental.pallas import tpu as pltpu
from jax.experimental.pallas import tpu_sc as plsc

SMOOTHING = 0.1
VOCAB = 32000
N_ROWS = 2048
EPS = SMOOTHING / (VOCAB - 2)
CONF = 1.0 - SMOOTHING
C_XLOGY = CONF * math.log(CONF) + SMOOTHING * math.log(EPS)

SC_CORES = 2
SC_SUBCORES = 16
SC_LANES = 16
NW = SC_CORES * SC_SUBCORES
B_PER_W = N_ROWS // NW


def _sc_gather_body(tgt_hbm, flat_hbm, out_hbm, tgt_v, idx_v, g_v, sem):
    wid = lax.axis_index("s") * SC_CORES + lax.axis_index("c")
    base = wid * B_PER_W
    pltpu.sync_copy(tgt_hbm.at[pl.ds(base, B_PER_W)], tgt_v)
    for j in range(B_PER_W // SC_LANES):
        t16 = tgt_v[pl.ds(j * SC_LANES, SC_LANES)]
        rows = (base + j * SC_LANES) + lax.iota(jnp.int32, SC_LANES)
        idx_v[pl.ds(j * SC_LANES, SC_LANES)] = rows * VOCAB + t16
    pltpu.async_copy(flat_hbm.at[idx_v], g_v, sem).wait()
    pltpu.sync_copy(g_v, out_hbm.at[pl.ds(base, B_PER_W)])


def _sc_gather(targets, flat_logits):
    mesh = plsc.VectorSubcoreMesh(core_axis_name="c", subcore_axis_name="s")
    run = functools.partial(
        pl.kernel,
        out_type=jax.ShapeDtypeStruct((N_ROWS,), jnp.float32),
        mesh=mesh,
        scratch_types=[
            pltpu.VMEM((B_PER_W,), jnp.int32),
            pltpu.VMEM((B_PER_W,), jnp.int32),
            pltpu.VMEM((B_PER_W,), jnp.float32),
            pltpu.SemaphoreType.DMA,
        ],
    )(_sc_gather_body)
    return run(targets, flat_logits)


ROW_BLK = 64


def _tc_loss_body(x_ref, t_ref, g_ref, out_ref, acc_ref):
    step = pl.program_id(0)

    @pl.when(step == 0)
    def _init():
        acc_ref[0] = 0.0
        acc_ref[1] = 0.0

    x = x_ref[...]
    m = jnp.max(x, axis=1, keepdims=True)
    s = jnp.sum(x, axis=1, keepdims=True)
    se = jnp.sum(jnp.exp(x - m), axis=1, keepdims=True)
    big_l = m + jnp.log(se)
    z = x[:, 0:1]
    g = g_ref[...]
    mask = (t_ref[...] != 0).astype(jnp.float32)
    contrib = (C_XLOGY
               - (CONF - EPS) * (g - big_l)
               - EPS * ((s - VOCAB * big_l) - (z - big_l)))
    acc_ref[0] += jnp.sum(mask * contrib)
    acc_ref[1] += jnp.sum(mask)

    @pl.when(step == pl.num_programs(0) - 1)
    def _fini():
        out_ref[...] = jnp.full((1, 1), acc_ref[0] / acc_ref[1], jnp.float32)


def _tc_loss(logits, targets2d, g2d, interpret=False):
    grid = N_ROWS // ROW_BLK
    return pl.pallas_call(
        _tc_loss_body,
        grid=(grid,),
        in_specs=[
            pl.BlockSpec((ROW_BLK, VOCAB), lambda i: (i, 0)),
            pl.BlockSpec((ROW_BLK, 1), lambda i: (i, 0)),
            pl.BlockSpec((ROW_BLK, 1), lambda i: (i, 0)),
        ],
        out_specs=pl.BlockSpec((1, 1), lambda i: (0, 0)),
        out_shape=jax.ShapeDtypeStruct((1, 1), jnp.float32),
        scratch_shapes=[pltpu.SMEM((2,), jnp.float32)],
        interpret=interpret,
    )(logits, targets2d, g2d)


def kernel(logits, targets):
    targets = targets.astype(jnp.int32)
    g = _sc_gather(targets, logits.reshape(-1))
    out = _tc_loss(logits,
                   targets.reshape(N_ROWS, 1),
                   g.reshape(N_ROWS, 1))
    return out[0, 0]


# speedup vs baseline: 2.8937x; 1.0458x over previous
import functools
import math

import jax
import jax.numpy as jnp
from jax import lax
from jax.experimental import pallas as pl
from jax.experimental.pallas import tpu as pltpu
from jax.experimental.pallas import tpu_sc as plsc

SMOOTHING = 0.1
VOCAB = 32000
N_ROWS = 2048
EPS = SMOOTHING / (VOCAB - 2)
CONF = 1.0 - SMOOTHING
C_XLOGY = CONF * math.log(CONF) + SMOOTHING * math.log(EPS)

SC_CORES = 2
SC_SUBCORES = 16
SC_LANES = 16
NW = SC_CORES * SC_SUBCORES
B_PER_W = N_ROWS // NW


def _sc_gather_body(tgt_hbm, flat_hbm, out_hbm, tgt_v, idx_v, g_v, sem):
    wid = lax.axis_index("s") * SC_CORES + lax.axis_index("c")
    base = wid * B_PER_W
    pltpu.sync_copy(tgt_hbm.at[pl.ds(base, B_PER_W)], tgt_v)
    for j in range(B_PER_W // SC_LANES):
        t16 = tgt_v[pl.ds(j * SC_LANES, SC_LANES)]
        rows = (base + j * SC_LANES) + lax.iota(jnp.int32, SC_LANES)
        idx_v[pl.ds(j * SC_LANES, SC_LANES)] = rows * VOCAB + t16
    pltpu.async_copy(flat_hbm.at[idx_v], g_v, sem).wait()
    pltpu.sync_copy(g_v, out_hbm.at[pl.ds(base, B_PER_W)])


def _sc_gather(targets, flat_logits):
    mesh = plsc.VectorSubcoreMesh(core_axis_name="c", subcore_axis_name="s")
    run = functools.partial(
        pl.kernel,
        out_type=jax.ShapeDtypeStruct((N_ROWS,), jnp.float32),
        mesh=mesh,
        scratch_types=[
            pltpu.VMEM((B_PER_W,), jnp.int32),
            pltpu.VMEM((B_PER_W,), jnp.int32),
            pltpu.VMEM((B_PER_W,), jnp.float32),
            pltpu.SemaphoreType.DMA,
        ],
    )(_sc_gather_body)
    return run(targets, flat_logits)


ROW_BLK = 64
COL_CHUNK = 256
N_CHUNKS = VOCAB // COL_CHUNK


def _tc_loss_body(x_ref, t_ref, g_ref, out_ref, acc_ref):
    step = pl.program_id(0)

    @pl.when(step == 0)
    def _init():
        acc_ref[0] = 0.0
        acc_ref[1] = 0.0

    m_acc = x_ref[:, 0:COL_CHUNK]
    s_acc = m_acc
    for c in range(1, N_CHUNKS):
        xc = x_ref[:, c * COL_CHUNK:(c + 1) * COL_CHUNK]
        m_acc = jnp.maximum(m_acc, xc)
        s_acc = s_acc + xc
    m = jnp.max(m_acc, axis=1, keepdims=True)
    s = jnp.sum(s_acc, axis=1, keepdims=True)

    e_acc = jnp.exp(x_ref[:, 0:COL_CHUNK] - m)
    for c in range(1, N_CHUNKS):
        e_acc = e_acc + jnp.exp(x_ref[:, c * COL_CHUNK:(c + 1) * COL_CHUNK] - m)
    lse = jnp.log(jnp.sum(e_acc, axis=1, keepdims=True))
    big_l = m + lse

    z = x_ref[:, 0:1]
    g = g_ref[...]
    mask = (t_ref[...] != 0).astype(jnp.float32)
    contrib = (C_XLOGY
               - (CONF - EPS) * (g - big_l)
               - EPS * ((s - VOCAB * big_l) - (z - big_l)))
    acc_ref[0] += jnp.sum(mask * contrib)
    acc_ref[1] += jnp.sum(mask)

    @pl.when(step == pl.num_programs(0) - 1)
    def _fini():
        out_ref[...] = jnp.full((1, 1), acc_ref[0] / acc_ref[1], jnp.float32)


def _tc_loss(logits, targets2d, g2d, interpret=False):
    grid = N_ROWS // ROW_BLK
    return pl.pallas_call(
        _tc_loss_body,
        grid=(grid,),
        in_specs=[
            pl.BlockSpec((ROW_BLK, VOCAB), lambda i: (i, 0)),
            pl.BlockSpec((ROW_BLK, 1), lambda i: (i, 0)),
            pl.BlockSpec((ROW_BLK, 1), lambda i: (i, 0)),
        ],
        out_specs=pl.BlockSpec((1, 1), lambda i: (0, 0)),
        out_shape=jax.ShapeDtypeStruct((1, 1), jnp.float32),
        scratch_shapes=[pltpu.SMEM((2,), jnp.float32)],
        interpret=interpret,
    )(logits, targets2d, g2d)


def kernel(logits, targets):
    targets = targets.astype(jnp.int32)
    g = _sc_gather(targets, logits.reshape(-1))
    out = _tc_loss(logits,
                   targets.reshape(N_ROWS, 1),
                   g.reshape(N_ROWS, 1))
    return out[0, 0]
